# baseline (speedup 1.0000x reference)
import functools

import jax
import jax.numpy as jnp
from jax import lax
from jax.experimental import pallas as pl
from jax.experimental.pallas import tpu as pltpu
from jax.experimental.pallas import tpu_sc as plsc

N_MODALITIES = 8
D_MODEL = 1024

NC = 2
NS = 16
NW = NC * NS

B = 4 * 8192
B_PER_W = B // NW
CHUNK = 32
N_CHUNKS = B_PER_W // CHUNK


def _sc_embedding_gather(ids_flat, emb):
    mesh = plsc.VectorSubcoreMesh(
        core_axis_name="c", subcore_axis_name="s", num_cores=NC, num_subcores=NS
    )

    @functools.partial(
        pl.kernel,
        out_type=jax.ShapeDtypeStruct((B, D_MODEL), jnp.float32),
        mesh=mesh,
        scratch_types=[
            pltpu.VMEM((B_PER_W,), jnp.int32),
            pltpu.VMEM((CHUNK, D_MODEL), jnp.float32),
            pltpu.VMEM((CHUNK, D_MODEL), jnp.float32),
            pltpu.SemaphoreType.DMA,
            pltpu.SemaphoreType.DMA,
        ],
    )
    def body(idx_hbm, emb_hbm, out_hbm, idx_v, rows0, rows1, gsem0, gsem1):
        wid = lax.axis_index("s") * NC + lax.axis_index("c")
        base = wid * B_PER_W
        pltpu.sync_copy(idx_hbm.at[pl.ds(base, B_PER_W)], idx_v)

        bufs = (rows0, rows1)
        sems = (gsem0, gsem1)

        pltpu.async_copy(emb_hbm.at[idx_v.at[pl.ds(0, CHUNK)]], rows0, gsem0)

        def step(g, _):
            slot = lax.rem(g, 2)

            def run(cur, nxt, cur_sem, nxt_sem):
                @pl.when(g + 1 < N_CHUNKS)
                def _():
                    pltpu.async_copy(
                        emb_hbm.at[idx_v.at[pl.ds((g + 1) * CHUNK, CHUNK)]],
                        nxt,
                        nxt_sem,
                    )

                pltpu.make_async_copy(
                    emb_hbm.at[idx_v.at[pl.ds(g * CHUNK, CHUNK)]], cur, cur_sem
                ).wait()
                pltpu.sync_copy(cur, out_hbm.at[pl.ds(base + g * CHUNK, CHUNK)])

            @pl.when(slot == 0)
            def _():
                run(bufs[0], bufs[1], sems[0], sems[1])

            @pl.when(slot == 1)
            def _():
                run(bufs[1], bufs[0], sems[1], sems[0])

            return 0

        lax.fori_loop(0, N_CHUNKS, step, 0)

    return body(ids_flat, emb)


def kernel(modality_ids, emb):
    ids_flat = modality_ids.reshape(-1).astype(jnp.int32)
    out = _sc_embedding_gather(ids_flat, emb)
    return out.reshape(modality_ids.shape + (emb.shape[1],))
